# baseline (speedup 1.0000x reference)
import functools

import jax
import jax.numpy as jnp
from jax import lax
from jax.experimental import pallas as pl
from jax.experimental.pallas import tpu as pltpu
from jax.experimental.pallas import tpu_sc as plsc

_N = 10000
_E = 320000
_H = 128
_NC = 2
_NS = 16
_NW = _NC * _NS
_CH = 128
_N_PAD = 12288
_E_PAD = 323584
_N_ACC = 10240
_BE = 2048
_BN = 2048

_f32 = jnp.float32
_bf16 = jnp.bfloat16


def _mesh():
    return plsc.VectorSubcoreMesh(
        core_axis_name="c", subcore_axis_name="s",
        num_cores=_NC, num_subcores=_NS)


def _sc_gather(table, idx3d, width, dtype):
    nw, k, _ = idx3d.shape
    nb, la = 4, 2
    ngroups = -(-k // nb)

    @functools.partial(
        pl.kernel,
        out_type=jax.ShapeDtypeStruct((nw * k * _CH, width), dtype),
        mesh=_mesh(),
        scratch_types=[
            pltpu.VMEM((k, _CH), jnp.int32),
            pltpu.VMEM((nb, _CH, width), dtype),
            pltpu.SemaphoreType.DMA((nb,)),
            pltpu.SemaphoreType.DMA((nb,)),
        ],
    )
    def gather_k(table_hbm, idx_hbm, out_hbm, idx_v, bufs, gsem, osem):
        c = lax.axis_index("c")
        s = lax.axis_index("s")
        w = s * _NC + c
        pltpu.sync_copy(idx_hbm.at[w], idx_v)
        base = w * (k * _CH)

        def g_start(j, b):
            pltpu.make_async_copy(
                table_hbm.at[idx_v.at[j]], bufs.at[b], gsem.at[b]).start()

        def g_wait(b):
            pltpu.make_async_copy(
                table_hbm.at[idx_v.at[0]], bufs.at[b], gsem.at[b]).wait()

        def o_start(j, b):
            pltpu.make_async_copy(
                bufs.at[b], out_hbm.at[pl.ds(base + j * _CH, _CH)],
                osem.at[b]).start()

        def o_wait(b):
            pltpu.make_async_copy(
                bufs.at[b], out_hbm.at[pl.ds(base, _CH)], osem.at[b]).wait()

        for j in range(min(la, k)):
            g_start(j, j % nb)

        def body(g, carry):
            for b in range(nb):
                j = g * nb + b
                jn = j + la
                bn = jn % nb

                @pl.when(jnp.logical_and(j >= la, jn < k))
                def _():
                    o_wait(bn)

                @pl.when(jn < k)
                def _():
                    g_start(jn, bn)

                @pl.when(j < k)
                def _():
                    g_wait(b)
                    o_start(j, b)
            return carry

        lax.fori_loop(0, ngroups, body, 0)
        for j in range(max(0, k - nb), k):
            o_wait(j % nb)

    return gather_k(table, idx3d)


def _sc_scatter_add(m, dst3d, zrows):
    nw, k, _ = dst3d.shape
    rows_per_s = _N_ACC // _NS
    nb, la = 2, 1
    ngroups = -(-k // nb)

    @functools.partial(
        pl.kernel,
        out_type=jax.ShapeDtypeStruct((_NC, _N_ACC, _H), _f32),
        mesh=_mesh(),
        scratch_types=[
            pltpu.VMEM((nb, _CH), jnp.int32),
            pltpu.VMEM((nb, _CH, _H), _f32),
            pltpu.VMEM_SHARED((_N_ACC, _H), _f32),
            pltpu.SemaphoreType.DMA((nb,)),
            pltpu.SemaphoreType.DMA((nb,)),
        ],
    )
    def scatter_k(m_hbm, dst_hbm, z_hbm, out_hbm, ibufs, mbufs, acc,
                  lsem, isem):
        c = lax.axis_index("c")
        s = lax.axis_index("s")
        w = s * _NC + c
        pltpu.sync_copy(z_hbm, acc.at[pl.ds(s * rows_per_s, rows_per_s)])
        plsc.subcore_barrier()
        base = w * (k * _CH)

        def l_start(j, b):
            pltpu.make_async_copy(
                m_hbm.at[pl.ds(base + j * _CH, _CH)], mbufs.at[b],
                lsem.at[b]).start()
            pltpu.make_async_copy(
                dst_hbm.at[w, j], ibufs.at[b], isem.at[b]).start()

        def l_wait(b):
            pltpu.make_async_copy(
                m_hbm.at[pl.ds(base, _CH)], mbufs.at[b], lsem.at[b]).wait()
            pltpu.make_async_copy(
                dst_hbm.at[w, 0], ibufs.at[b], isem.at[b]).wait()

        for j in range(min(la, k)):
            l_start(j, j % nb)

        def body(g, carry):
            for b in range(nb):
                j = g * nb + b
                jn = j + la

                @pl.when(jn < k)
                def _():
                    l_start(jn, jn % nb)

                @pl.when(j < k)
                def _():
                    l_wait(b)
                    pltpu.sync_copy(mbufs.at[b], acc.at[ibufs.at[b]], add=True)
            return carry

        lax.fori_loop(0, ngroups, body, 0)
        plsc.subcore_barrier()
        pltpu.sync_copy(acc.at[pl.ds(s * rows_per_s, rows_per_s)],
                        out_hbm.at[c, pl.ds(s * rows_per_s, rows_per_s)])

    return scatter_k(m, dst3d, zrows)


def _ln(h):
    mu = jnp.mean(h, axis=-1, keepdims=True)
    d = h - mu
    var = jnp.mean(d * d, axis=-1, keepdims=True)
    return d * lax.rsqrt(var + 1e-5)


def _b16(x):
    return x.astype(_bf16)


def _dot(a16, b16):
    return jnp.dot(a16, b16, preferred_element_type=_f32)


def _full(spec_shape):
    return pl.BlockSpec(spec_shape, lambda i: tuple(0 for _ in spec_shape))


def _w16(params):
    return tuple((w.astype(_bf16), b.reshape(1, -1)) for w, b in params)


def _mlp3_body(x_ref, w1, b1, w2, b2, w3, b3, o_ref, *, layernorm):
    h = jnp.maximum(_dot(_b16(x_ref[...]), w1[...]) + b1[...], 0.0)
    h = jnp.maximum(_dot(_b16(h), w2[...]) + b2[...], 0.0)
    o_ref[...] = _dot(_b16(h), w3[...]) + b3[...]


def _mlp3(x, params, *, layernorm, block, out_dim):
    n, din = x.shape
    (w1, b1), (w2, b2), (w3, b3) = _w16(params)
    grid = (n // block,)
    return pl.pallas_call(
        functools.partial(_mlp3_body, layernorm=layernorm),
        grid=grid,
        in_specs=[
            pl.BlockSpec((block, din), lambda i: (i, 0)),
            _full(w1.shape), _full(b1.shape),
            _full(w2.shape), _full(b2.shape),
            _full(w3.shape), _full(b3.shape),
        ],
        out_specs=pl.BlockSpec((block, out_dim), lambda i: (i, 0)),
        out_shape=jax.ShapeDtypeStruct((n, out_dim), _f32),
    )(x, w1, b1, w2, b2, w3, b3)


def _nodein_body(g_ref, pos_ref, w1x, b1, w2, b2, w3, b3, o_ref):
    xcat = g_ref[...] + pos_ref[...]
    h = jnp.maximum(_dot(_b16(xcat), w1x[...]) + b1[...], 0.0)
    h = jnp.maximum(_dot(_b16(h), w2[...]) + b2[...], 0.0)
    o_ref[...] = _dot(_b16(h), w3[...]) + b3[...]


def _node_in(g_emb, pos128, params):
    (w1, b1), (w2, b2), (w3, b3) = _w16(params)
    w1x = jnp.pad(w1, ((0, _H - w1.shape[0]), (0, 0)))
    grid = (_N_PAD // _BN,)
    return pl.pallas_call(
        _nodein_body,
        grid=grid,
        in_specs=[
            pl.BlockSpec((_BN, _H), lambda i: (i, 0)),
            pl.BlockSpec((_BN, _H), lambda i: (i, 0)),
            _full((_H, _H)), _full(b1.shape),
            _full((_H, _H)), _full(b2.shape),
            _full((_H, _H)), _full(b3.shape),
        ],
        out_specs=pl.BlockSpec((_BN, _H), lambda i: (i, 0)),
        out_shape=jax.ShapeDtypeStruct((_N_PAD, _H), _f32),
    )(g_emb, pos128, w1x, b1, w2, b2, w3, b3)


def _edge_body(ga_ref, gb_ref, ef_ref, nd_ref, w1, b1, w2, b2,
               w3, b3, wd1, bd1, wd2, bd2, m_ref, efo_ref):
    xcat = jnp.concatenate(
        [_b16(ga_ref[0]), _b16(gb_ref[0]), _b16(ef_ref[...])], axis=-1)
    h = _dot(xcat, w1[...]) + b1[...]
    h = jnp.maximum(h, 0.0)
    h = jnp.maximum(_dot(_b16(h), w2[...]) + b2[...], 0.0)
    h = _dot(_b16(h), w3[...]) + b3[...]
    m_ref[...] = h
    nd = _b16(nd_ref[0]).astype(_f32)
    hd = jnp.maximum(nd * wd1[...].astype(_f32) + bd1[...], 0.0)
    efo_ref[...] = _dot(_b16(hd), wd2[...]) + bd2[...]


def _edge_mlp(g, ef, nd3, p_edge, p_dist):
    (w1, b1), (w2, b2), (w3, b3) = _w16(p_edge)
    (wd1, bd1), (wd2, bd2) = _w16(p_dist)
    grid = (_E_PAD // _BE,)
    return pl.pallas_call(
        _edge_body,
        grid=grid,
        in_specs=[
            pl.BlockSpec((1, _BE, _H), lambda i: (0, i, 0)),
            pl.BlockSpec((1, _BE, _H), lambda i: (1, i, 0)),
            pl.BlockSpec((_BE, _H), lambda i: (i, 0)),
            pl.BlockSpec((1, _BE, 1), lambda i: (i, 0, 0)),
            _full((3 * _H, _H)), _full(b1.shape),
            _full((_H, _H)), _full(b2.shape),
            _full((_H, _H)), _full(b3.shape),
            _full((1, _H)), _full(bd1.shape),
            _full((_H, _H)), _full(bd2.shape),
        ],
        out_specs=[
            pl.BlockSpec((_BE, _H), lambda i: (i, 0)),
            pl.BlockSpec((_BE, _H), lambda i: (i, 0)),
        ],
        out_shape=[
            jax.ShapeDtypeStruct((_E_PAD, _H), _f32),
            jax.ShapeDtypeStruct((_E_PAD, _H), _f32),
        ],
    )(g, g, ef, nd3, w1, b1,
      w2, b2, w3, b3, wd1, bd1, wd2, bd2)


def _node_body(nf_ref, p0_ref, p1_ref, w1, b1, w2, b2, w3, b3, o_ref):
    aggr = p0_ref[0] + p1_ref[0]
    xcat = jnp.concatenate([_b16(nf_ref[...]), _b16(aggr)], axis=-1)
    h = jnp.maximum(_dot(xcat, w1[...]) + b1[...], 0.0)
    h = jnp.maximum(_dot(_b16(h), w2[...]) + b2[...], 0.0)
    o_ref[...] = _dot(_b16(h), w3[...]) + b3[...]


def _node_update(nf, partials, p_node):
    (w1, b1), (w2, b2), (w3, b3) = _w16(p_node)
    grid = (_N_PAD // _BN,)
    nblk = _N_ACC // _BN - 1
    return pl.pallas_call(
        _node_body,
        grid=grid,
        in_specs=[
            pl.BlockSpec((_BN, _H), lambda i: (i, 0)),
            pl.BlockSpec((1, _BN, _H), lambda i: (0, jnp.minimum(i, nblk), 0)),
            pl.BlockSpec((1, _BN, _H), lambda i: (1, jnp.minimum(i, nblk), 0)),
            _full((2 * _H, _H)), _full(b1.shape),
            _full((_H, _H)), _full(b2.shape),
            _full((_H, _H)), _full(b3.shape),
        ],
        out_specs=pl.BlockSpec((_BN, _H), lambda i: (i, 0)),
        out_shape=jax.ShapeDtypeStruct((_N_PAD, _H), _f32),
    )(nf, partials, partials, w1, b1, w2, b2, w3, b3)


def kernel(x, pos, edge_index, edge_attr, node_dist, params):
    src = edge_index[0].astype(jnp.int32)
    dst = edge_index[1].astype(jnp.int32)
    ep = _E_PAD - _E
    np_ = _N_PAD - _N

    dst_g = jnp.concatenate([dst, jnp.zeros((ep,), jnp.int32)])
    src_g = jnp.concatenate([src, jnp.zeros((ep,), jnp.int32)])
    idx_comb = jnp.concatenate([dst_g, src_g]).reshape(_NW, -1, _CH)
    dst_s = jnp.concatenate(
        [dst, jnp.full((ep,), _N, jnp.int32)]).reshape(_NW, -1, _CH)
    x_pad = jnp.concatenate(
        [x.astype(jnp.int32), jnp.zeros((np_,), jnp.int32)]).reshape(
            _NW, -1, _CH)
    pos128 = jnp.pad(pos.astype(_f32), ((0, np_), (16, 128 - 37)))
    ea_pad = jnp.pad(edge_attr.astype(_f32), ((0, ep), (0, 0)))
    nd3 = jnp.pad(node_dist.astype(_f32), (0, ep)).reshape(
        _E_PAD // _BE, _BE, 1)
    zrows = jnp.zeros((_N_ACC // _NS, _H), _f32)

    embed_pad = jnp.pad(params['embed'].astype(_f32), ((0, 7), (0, 112)))
    g_emb = _sc_gather(embed_pad, x_pad, _H, _f32)
    nf = _ln(_node_in(g_emb, pos128, params['node_in']))
    ef = _ln(_mlp3(ea_pad, params['edge_in'], layernorm=True, block=_BE,
                   out_dim=_H))

    for p in params['layers']:
        g = _sc_gather(nf, idx_comb, _H, _f32)
        g = g.reshape(2, _E_PAD, _H)
        h3, wgt = _edge_mlp(g, ef, nd3, p['edge_mlp'], p['dist'])
        m = _ln(h3) * wgt
        ef = ef + m
        partials = _sc_scatter_add(m, dst_s, zrows)
        nf = nf + _ln(_node_update(nf, partials, p['node_mlp']))

    out = _mlp3(nf, params['node_out'], layernorm=False, block=_BN, out_dim=3)
    return out[:_N]
